# Initial kernel scaffold; baseline (speedup 1.0000x reference)
import jax
import jax.numpy as jnp
from jax.experimental import pallas as pl


def kernel(node_features, edge_index, edge_features, W_msg, b_msg, W_ih, W_hh, b_ih, b_hh, W_c1, b_c1, W_c2, b_c2):
    raise NotImplementedError("write your pallas kernel here")


# speedup vs baseline: 2.6457x; 2.6457x over previous
import functools

import jax
import jax.numpy as jnp
from jax import lax
from jax.experimental import pallas as pl
from jax.experimental.pallas import tpu as pltpu
from jax.experimental.pallas import tpu_sc as plsc

_NC = 2
_NS = 16
_NW = _NC * _NS
_LANES = 16


def _gelu_exp(x):
    u = 0.7978845608028654 * (x + 0.044715 * x * x * x)
    e = jnp.exp(2.0 * u)
    return 0.5 * x * (1.0 + (e - 1.0) / (e + 1.0))


def _node_tables_body(nf_ref, w0_ref, w1_ref, a_ref, b_ref):
    x = nf_ref[...]
    a_ref[...] = jnp.dot(x, w0_ref[...], preferred_element_type=jnp.float32)
    b_ref[...] = jnp.dot(x, w1_ref[...], preferred_element_type=jnp.float32)


def _edge_tables_body(ef_ref, wce_ref, bce_ref, wr_ref, br_ref, ce_ref, r_ref):
    x = ef_ref[...]
    ce_ref[...] = jnp.dot(x, wce_ref[...], preferred_element_type=jnp.float32) + bce_ref[...]
    r_ref[...] = jnp.dot(x, wr_ref[...], preferred_element_type=jnp.float32) + br_ref[...]


def _gru_body(a0_ref, a1_ref, nf_ref, wih_ref, whh_ref, bih_ref, bhh_ref,
              wp_ref, wq_ref, p_ref, q_ref):
    d = nf_ref.shape[1]
    agg = a0_ref[...] + a1_ref[...]
    nf = nf_ref[...]
    gi = lax.dot_general(agg, wih_ref[...], (((1,), (1,)), ((), ())),
                         preferred_element_type=jnp.float32) + bih_ref[...]
    gh = lax.dot_general(nf, whh_ref[...], (((1,), (1,)), ((), ())),
                         preferred_element_type=jnp.float32) + bhh_ref[...]
    r = jax.nn.sigmoid(gi[:, :d] + gh[:, :d])
    z = jax.nn.sigmoid(gi[:, d:2 * d] + gh[:, d:2 * d])
    n = jnp.tanh(gi[:, 2 * d:] + r * gh[:, 2 * d:])
    nf_up = (1.0 - z) * n + z * nf
    p_ref[...] = jnp.dot(nf_up, wp_ref[...], preferred_element_type=jnp.float32)
    q_ref[...] = jnp.dot(nf_up, wq_ref[...], preferred_element_type=jnp.float32)


def _cls_out_body(h_ref, w_ref, b_ref, o_ref):
    o_ref[...] = jnp.dot(h_ref[...], w_ref[...],
                         preferred_element_type=jnp.float32) + b_ref[...]


def _make_msg_agg(n, d, e, chunk):
    ew = e // _NW
    nchunks = ew // chunk
    zrows = 64
    n_pad = -(-n // (_NS * zrows)) * (_NS * zrows)
    rows_per_tile = n_pad // _NS
    mesh = plsc.VectorSubcoreMesh(core_axis_name="c", subcore_axis_name="s")

    @functools.partial(
        pl.kernel,
        mesh=mesh,
        out_type=jax.ShapeDtypeStruct((2 * n_pad, d), jnp.float32),
        scratch_types=[
            pltpu.VMEM((chunk,), jnp.int32),
            pltpu.VMEM((chunk,), jnp.int32),
            pltpu.VMEM((chunk, d), jnp.float32),
            pltpu.VMEM((chunk, d), jnp.float32),
            pltpu.VMEM((chunk, d), jnp.float32),
            pltpu.VMEM((zrows, d), jnp.float32),
            pltpu.VMEM_SHARED((n_pad, d), jnp.float32),
            pltpu.SemaphoreType.DMA,
            pltpu.SemaphoreType.DMA,
            pltpu.SemaphoreType.DMA,
        ],
    )
    def msg_agg(a_hbm, b_hbm, ce_hbm, src_hbm, tgt_hbm, out_hbm,
                sidx, tidx, buf_a, buf_b, buf_c, zbuf, agg_sh,
                sem_a, sem_b, sem_c):
        cid = lax.axis_index("c")
        sid = lax.axis_index("s")
        base = (cid * _NS + sid) * ew

        def zrow(i, carry):
            for c in range(d // _LANES):
                zbuf[i, pl.ds(_LANES * c, _LANES)] = jnp.zeros((_LANES,), jnp.float32)
            return carry

        lax.fori_loop(0, zrows, zrow, 0)
        for t in range(rows_per_tile // zrows):
            pltpu.sync_copy(zbuf, agg_sh.at[pl.ds(sid * rows_per_tile + t * zrows, zrows)])
        plsc.subcore_barrier()

        def do_chunk(j, carry):
            off = base + j * chunk
            pltpu.sync_copy(src_hbm.at[pl.ds(off, chunk)], sidx)
            pltpu.sync_copy(tgt_hbm.at[pl.ds(off, chunk)], tidx)
            cp_a = pltpu.async_copy(a_hbm.at[sidx], buf_a, sem_a)
            cp_b = pltpu.async_copy(b_hbm.at[tidx], buf_b, sem_b)
            cp_c = pltpu.async_copy(ce_hbm.at[pl.ds(off, chunk)], buf_c, sem_c)
            cp_a.wait()
            cp_b.wait()
            cp_c.wait()

            def row(r, inner):
                for c in range(d // _LANES):
                    s = pl.ds(_LANES * c, _LANES)
                    buf_a[r, s] = _gelu_exp(buf_a[r, s] + buf_b[r, s] + buf_c[r, s])
                return inner

            lax.fori_loop(0, chunk, row, 0)
            pltpu.sync_copy(buf_a, agg_sh.at[tidx], add=True)
            return carry

        lax.fori_loop(0, nchunks, do_chunk, 0)
        plsc.subcore_barrier()
        for t in range(rows_per_tile // zrows):
            sl = pl.ds(sid * rows_per_tile + t * zrows, zrows)
            dst = pl.ds(cid * n_pad + sid * rows_per_tile + t * zrows, zrows)
            pltpu.sync_copy(agg_sh.at[sl], out_hbm.at[dst])

    msg_agg.n_pad = n_pad
    return msg_agg


def _make_cls_gather(n, de, e, chunk):
    ew = e // _NW
    nchunks = ew // chunk
    mesh = plsc.VectorSubcoreMesh(core_axis_name="c", subcore_axis_name="s")

    @functools.partial(
        pl.kernel,
        mesh=mesh,
        out_type=jax.ShapeDtypeStruct((e, de), jnp.float32),
        scratch_types=[
            pltpu.VMEM((chunk,), jnp.int32),
            pltpu.VMEM((chunk,), jnp.int32),
            pltpu.VMEM((chunk, de), jnp.float32),
            pltpu.VMEM((chunk, de), jnp.float32),
            pltpu.VMEM((chunk, de), jnp.float32),
            pltpu.SemaphoreType.DMA,
            pltpu.SemaphoreType.DMA,
            pltpu.SemaphoreType.DMA,
        ],
        compiler_params=pltpu.CompilerParams(use_tc_tiling_on_sc=False),
    )
    def cls_gather(p_hbm, q_hbm, r_hbm, src_hbm, tgt_hbm, h_hbm,
                   sidx, tidx, buf_p, buf_q, buf_r, sem_p, sem_q, sem_r):
        cid = lax.axis_index("c")
        sid = lax.axis_index("s")
        base = (cid * _NS + sid) * ew

        def do_chunk(j, carry):
            off = base + j * chunk
            pltpu.sync_copy(src_hbm.at[pl.ds(off, chunk)], sidx)
            pltpu.sync_copy(tgt_hbm.at[pl.ds(off, chunk)], tidx)
            cp_p = pltpu.async_copy(p_hbm.at[sidx], buf_p, sem_p)
            cp_q = pltpu.async_copy(q_hbm.at[tidx], buf_q, sem_q)
            cp_r = pltpu.async_copy(r_hbm.at[pl.ds(off, chunk)], buf_r, sem_r)
            cp_p.wait()
            cp_q.wait()
            cp_r.wait()

            def rows8(r8, inner):
                for u in range(8):
                    r = r8 * 8 + u
                    s = pl.ds(0, _LANES)
                    buf_p[r, s] = _gelu_exp(buf_p[r, s] + buf_q[r, s] + buf_r[r, s])
                return inner

            lax.fori_loop(0, chunk // 8, rows8, 0)
            pltpu.sync_copy(buf_p, h_hbm.at[pl.ds(off, chunk)])
            return carry

        lax.fori_loop(0, nchunks, do_chunk, 0)

    return cls_gather


def kernel(node_features, edge_index, edge_features, W_msg, b_msg,
           W_ih, W_hh, b_ih, b_hh, W_c1, b_c1, W_c2, b_c2):
    n, d = node_features.shape
    e = edge_index.shape[1]
    de = edge_features.shape[1]
    c = W_c2.shape[1]
    assert e % _NW == 0 and (e // _NW) % 80 == 0 and n % _NS == 0

    src = edge_index[0]
    tgt = edge_index[1]

    nblk = 2000
    a_tab, b_tab = pl.pallas_call(
        _node_tables_body,
        grid=(n // nblk,),
        in_specs=[
            pl.BlockSpec((nblk, d), lambda i: (i, 0)),
            pl.BlockSpec((d, d), lambda i: (0, 0)),
            pl.BlockSpec((d, d), lambda i: (0, 0)),
        ],
        out_specs=[
            pl.BlockSpec((nblk, d), lambda i: (i, 0)),
            pl.BlockSpec((nblk, d), lambda i: (i, 0)),
        ],
        out_shape=[
            jax.ShapeDtypeStruct((n, d), jnp.float32),
            jax.ShapeDtypeStruct((n, d), jnp.float32),
        ],
    )(node_features, W_msg[:d], W_msg[d:2 * d])

    eblk = 8000
    ce_tab, r_tab = pl.pallas_call(
        _edge_tables_body,
        grid=(e // eblk,),
        in_specs=[
            pl.BlockSpec((eblk, de), lambda i: (i, 0)),
            pl.BlockSpec((de, d), lambda i: (0, 0)),
            pl.BlockSpec((1, d), lambda i: (0, 0)),
            pl.BlockSpec((de, de), lambda i: (0, 0)),
            pl.BlockSpec((1, de), lambda i: (0, 0)),
        ],
        out_specs=[
            pl.BlockSpec((eblk, d), lambda i: (i, 0)),
            pl.BlockSpec((eblk, de), lambda i: (i, 0)),
        ],
        out_shape=[
            jax.ShapeDtypeStruct((e, d), jnp.float32),
            jax.ShapeDtypeStruct((e, de), jnp.float32),
        ],
    )(edge_features, W_msg[2 * d:], b_msg.reshape(1, d),
      W_c1[2 * d:], b_c1.reshape(1, de))

    msg_agg = _make_msg_agg(n, d, e, chunk=80)
    agg2 = msg_agg(a_tab, b_tab, ce_tab, src, tgt)
    agg2 = agg2.reshape(2, msg_agg.n_pad, d)[:, :n, :]

    p_tab, q_tab = pl.pallas_call(
        _gru_body,
        grid=(n // nblk,),
        in_specs=[
            pl.BlockSpec((nblk, d), lambda i: (i, 0)),
            pl.BlockSpec((nblk, d), lambda i: (i, 0)),
            pl.BlockSpec((nblk, d), lambda i: (i, 0)),
            pl.BlockSpec((3 * d, d), lambda i: (0, 0)),
            pl.BlockSpec((3 * d, d), lambda i: (0, 0)),
            pl.BlockSpec((1, 3 * d), lambda i: (0, 0)),
            pl.BlockSpec((1, 3 * d), lambda i: (0, 0)),
            pl.BlockSpec((d, de), lambda i: (0, 0)),
            pl.BlockSpec((d, de), lambda i: (0, 0)),
        ],
        out_specs=[
            pl.BlockSpec((nblk, de), lambda i: (i, 0)),
            pl.BlockSpec((nblk, de), lambda i: (i, 0)),
        ],
        out_shape=[
            jax.ShapeDtypeStruct((n, de), jnp.float32),
            jax.ShapeDtypeStruct((n, de), jnp.float32),
        ],
    )(agg2[0], agg2[1], node_features, W_ih, W_hh,
      b_ih.reshape(1, 3 * d), b_hh.reshape(1, 3 * d),
      W_c1[:d], W_c1[d:2 * d])

    h = _make_cls_gather(n, de, e, chunk=80)(p_tab, q_tab, r_tab, src, tgt)

    out = pl.pallas_call(
        _cls_out_body,
        grid=(e // eblk,),
        in_specs=[
            pl.BlockSpec((eblk, de), lambda i: (i, 0)),
            pl.BlockSpec((de, c), lambda i: (0, 0)),
            pl.BlockSpec((1, c), lambda i: (0, 0)),
        ],
        out_specs=pl.BlockSpec((eblk, c), lambda i: (i, 0)),
        out_shape=jax.ShapeDtypeStruct((e, c), jnp.float32),
    )(h, W_c2, b_c2.reshape(1, c))

    return out


# speedup vs baseline: 3.5530x; 1.3430x over previous
import functools

import jax
import jax.numpy as jnp
from jax import lax
from jax.experimental import pallas as pl
from jax.experimental.pallas import tpu as pltpu
from jax.experimental.pallas import tpu_sc as plsc

_NC = 2
_NS = 16
_NW = _NC * _NS
_LANES = 16


def _gelu_exp(x):
    e = jnp.exp(1.5957691216057308 * (x + 0.044715 * x * x * x))
    return x * e / (e + 1.0)


def _node_tables_body(nf_ref, w0_ref, w1_ref, a_ref, b_ref):
    x = nf_ref[...]
    a_ref[...] = jnp.dot(x, w0_ref[...], preferred_element_type=jnp.float32)
    b_ref[...] = jnp.dot(x, w1_ref[...], preferred_element_type=jnp.float32)


def _edge_tables_body(ef_ref, wce_ref, bce_ref, wr_ref, br_ref, ce_ref, r_ref):
    x = ef_ref[...]
    ce_ref[...] = jnp.dot(x, wce_ref[...], preferred_element_type=jnp.float32) + bce_ref[...]
    r_ref[...] = jnp.dot(x, wr_ref[...], preferred_element_type=jnp.float32) + br_ref[...]


def _gru_body(a0_ref, a1_ref, nf_ref, wih_ref, whh_ref, bih_ref, bhh_ref,
              wp_ref, wq_ref, p_ref, q_ref):
    d = nf_ref.shape[1]
    agg = a0_ref[...] + a1_ref[...]
    nf = nf_ref[...]
    gi = lax.dot_general(agg, wih_ref[...], (((1,), (1,)), ((), ())),
                         preferred_element_type=jnp.float32) + bih_ref[...]
    gh = lax.dot_general(nf, whh_ref[...], (((1,), (1,)), ((), ())),
                         preferred_element_type=jnp.float32) + bhh_ref[...]
    r = jax.nn.sigmoid(gi[:, :d] + gh[:, :d])
    z = jax.nn.sigmoid(gi[:, d:2 * d] + gh[:, d:2 * d])
    n = jnp.tanh(gi[:, 2 * d:] + r * gh[:, 2 * d:])
    nf_up = (1.0 - z) * n + z * nf
    p_ref[...] = jnp.dot(nf_up, wp_ref[...], preferred_element_type=jnp.float32)
    q_ref[...] = jnp.dot(nf_up, wq_ref[...], preferred_element_type=jnp.float32)


def _cls_out_body(h_ref, w_ref, b_ref, o_ref):
    o_ref[...] = jnp.dot(h_ref[...], w_ref[...],
                         preferred_element_type=jnp.float32) + b_ref[...]


def _make_msg_agg(n, d, e, chunk):
    ew = e // _NW
    nchunks = ew // chunk
    npairs = nchunks // 2
    zrows = chunk
    n_pad = -(-n // (_NS * zrows)) * (_NS * zrows)
    rows_per_tile = n_pad // _NS
    mesh = plsc.VectorSubcoreMesh(core_axis_name="c", subcore_axis_name="s")

    assert nchunks % 4 == 2 and nchunks >= 6
    buf_t = pltpu.VMEM((chunk, d), jnp.float32)

    @functools.partial(
        pl.kernel,
        mesh=mesh,
        out_type=jax.ShapeDtypeStruct((2 * n_pad, d), jnp.float32),
        scratch_types=[
            pltpu.VMEM((4, chunk), jnp.int32),
            pltpu.VMEM((4, chunk), jnp.int32),
            buf_t, buf_t, buf_t,
            buf_t, buf_t, buf_t,
            pltpu.VMEM_SHARED((n_pad, d), jnp.float32),
            pltpu.SemaphoreType.DMA,
            pltpu.SemaphoreType.DMA,
            pltpu.SemaphoreType.DMA,
            pltpu.SemaphoreType.DMA,
            pltpu.SemaphoreType.DMA,
            pltpu.SemaphoreType.DMA,
        ],
    )
    def msg_agg(a_hbm, b_hbm, ce_hbm, idx_hbm, out_hbm,
                iba, ibb, ga0, gb0, gc0, ga1, gb1, gc1, agg_sh,
                sa0, sb0, sc0, sa1, sb1, sc1):
        cid = lax.axis_index("c")
        sid = lax.axis_index("s")
        wid = cid * _NS + sid
        base = wid * ew
        pbase = wid * npairs
        gsets = ((ga0, gb0, gc0, sa0, sb0, sc0), (ga1, gb1, gc1, sa1, sb1, sc1))

        def fire(st, ib, half, j):
            ga, gb, gc, sa, sb, sc = gsets[st]
            pltpu.async_copy(a_hbm.at[ib.at[2 * half]], ga, sa)
            pltpu.async_copy(b_hbm.at[ib.at[2 * half + 1]], gb, sb)
            pltpu.async_copy(ce_hbm.at[pl.ds(base + j * chunk, chunk)], gc, sc)

        def wait(st, ib, half, j):
            ga, gb, gc, sa, sb, sc = gsets[st]
            pltpu.make_async_copy(a_hbm.at[ib.at[2 * half]], ga, sa).wait()
            pltpu.make_async_copy(b_hbm.at[ib.at[2 * half + 1]], gb, sb).wait()
            pltpu.make_async_copy(ce_hbm.at[pl.ds(base + j * chunk, chunk)], gc, sc).wait()

        def compute_scatter(st, ib, half):
            ga, gb, gc = gsets[st][:3]

            def row(r, carry):
                for c in range(d // _LANES):
                    s = pl.ds(_LANES * c, _LANES)
                    ga[r, s] = _gelu_exp(ga[r, s] + gb[r, s] + gc[r, s])
                return carry

            lax.fori_loop(0, chunk, row, 0)
            pltpu.sync_copy(ga, agg_sh.at[ib.at[2 * half + 1]], add=True)

        def zrow(i, carry):
            for c in range(d // _LANES):
                ga0[i, pl.ds(_LANES * c, _LANES)] = jnp.zeros((_LANES,), jnp.float32)
            return carry

        lax.fori_loop(0, chunk, zrow, 0)
        for t in range(rows_per_tile // zrows):
            pltpu.sync_copy(ga0, agg_sh.at[pl.ds(sid * rows_per_tile + t * zrows, zrows)])
        plsc.subcore_barrier()

        pltpu.sync_copy(idx_hbm.at[pbase], iba)
        fire(0, iba, 0, 0)

        def body(k, carry):
            a = 4 * k
            pltpu.sync_copy(idx_hbm.at[pbase + 2 * k + 1], ibb)
            fire(1, iba, 1, a + 1)
            wait(0, iba, 0, a)
            compute_scatter(0, iba, 0)
            fire(0, ibb, 0, a + 2)
            wait(1, iba, 1, a + 1)
            compute_scatter(1, iba, 1)
            pltpu.sync_copy(idx_hbm.at[pbase + 2 * k + 2], iba)
            fire(1, ibb, 1, a + 3)
            wait(0, ibb, 0, a + 2)
            compute_scatter(0, ibb, 0)
            fire(0, iba, 0, a + 4)
            wait(1, ibb, 1, a + 3)
            compute_scatter(1, ibb, 1)
            return carry

        lax.fori_loop(0, (npairs - 1) // 2, body, 0)
        last = nchunks - 2
        fire(1, iba, 1, last + 1)
        wait(0, iba, 0, last)
        compute_scatter(0, iba, 0)
        wait(1, iba, 1, last + 1)
        compute_scatter(1, iba, 1)

        plsc.subcore_barrier()
        for t in range(rows_per_tile // zrows):
            sl = pl.ds(sid * rows_per_tile + t * zrows, zrows)
            dst = pl.ds(cid * n_pad + sid * rows_per_tile + t * zrows, zrows)
            pltpu.sync_copy(agg_sh.at[sl], out_hbm.at[dst])

    msg_agg.n_pad = n_pad
    return msg_agg


def _make_cls_gather(n, de, e, chunk):
    ew = e // _NW
    nchunks = ew // chunk
    assert nchunks % 2 == 1 and nchunks >= 3
    mesh = plsc.VectorSubcoreMesh(core_axis_name="c", subcore_axis_name="s")
    buf_t = pltpu.VMEM((chunk, de), jnp.float32)

    @functools.partial(
        pl.kernel,
        mesh=mesh,
        out_type=jax.ShapeDtypeStruct((e, de), jnp.float32),
        scratch_types=[
            pltpu.VMEM((nchunks, chunk), jnp.int32),
            pltpu.VMEM((nchunks, chunk), jnp.int32),
            buf_t, buf_t, buf_t,
            buf_t, buf_t, buf_t,
            buf_t, buf_t,
            pltpu.SemaphoreType.DMA,
            pltpu.SemaphoreType.DMA,
            pltpu.SemaphoreType.DMA,
            pltpu.SemaphoreType.DMA,
            pltpu.SemaphoreType.DMA,
            pltpu.SemaphoreType.DMA,
            pltpu.SemaphoreType.DMA,
            pltpu.SemaphoreType.DMA,
        ],
        compiler_params=pltpu.CompilerParams(use_tc_tiling_on_sc=False),
    )
    def cls_gather(p_hbm, q_hbm, r_hbm, src_hbm, tgt_hbm, h_hbm,
                   sslab, tslab, gp0, gq0, gr0, gp1, gq1, gr1, m0, m1,
                   sp0, sq0, sr0, sp1, sq1, sr1, sw0, sw1):
        cid = lax.axis_index("c")
        sid = lax.axis_index("s")
        wid = cid * _NS + sid
        base = wid * ew
        gsets = ((gp0, gq0, gr0, sp0, sq0, sr0), (gp1, gq1, gr1, sp1, sq1, sr1))

        def fire_gathers(st, j):
            gp, gq, gr, sp, sq, sr = gsets[st]
            pltpu.async_copy(p_hbm.at[sslab.at[j]], gp, sp)
            pltpu.async_copy(q_hbm.at[tslab.at[j]], gq, sq)
            pltpu.async_copy(r_hbm.at[pl.ds(base + j * chunk, chunk)], gr, sr)

        def wait_gathers(st, j):
            gp, gq, gr, sp, sq, sr = gsets[st]
            pltpu.make_async_copy(p_hbm.at[sslab.at[j]], gp, sp).wait()
            pltpu.make_async_copy(q_hbm.at[tslab.at[j]], gq, sq).wait()
            pltpu.make_async_copy(r_hbm.at[pl.ds(base + j * chunk, chunk)], gr, sr).wait()

        def compute(st, m):
            gp, gq, gr = gsets[st][:3]

            def rows8(r8, carry):
                for u in range(8):
                    r = r8 * 8 + u
                    s = pl.ds(0, _LANES)
                    m[r, s] = _gelu_exp(gp[r, s] + gq[r, s] + gr[r, s])
                return carry

            lax.fori_loop(0, chunk // 8, rows8, 0)

        pltpu.sync_copy(src_hbm.at[wid], sslab)
        pltpu.sync_copy(tgt_hbm.at[wid], tslab)

        pltpu.async_copy(m0, h_hbm.at[pl.ds(base, chunk)], sw0)
        pltpu.async_copy(m1, h_hbm.at[pl.ds(base + chunk, chunk)], sw1)
        fire_gathers(0, 0)

        def wait_write(m, j, sem):
            pltpu.make_async_copy(m, h_hbm.at[pl.ds(base + j * chunk, chunk)], sem).wait()

        def body(j2, carry):
            a = 2 * j2
            b = a + 1
            fire_gathers(1, b)
            wait_gathers(0, a)
            wait_write(m0, a, sw0)
            compute(0, m0)
            pltpu.async_copy(m0, h_hbm.at[pl.ds(base + a * chunk, chunk)], sw0)
            fire_gathers(0, a + 2)
            wait_gathers(1, b)
            wait_write(m1, b, sw1)
            compute(1, m1)
            pltpu.async_copy(m1, h_hbm.at[pl.ds(base + b * chunk, chunk)], sw1)
            return carry

        lax.fori_loop(0, nchunks // 2, body, 0)
        last = nchunks - 1
        wait_gathers(0, last)
        wait_write(m0, last, sw0)
        compute(0, m0)
        pltpu.async_copy(m0, h_hbm.at[pl.ds(base + last * chunk, chunk)], sw0)
        wait_write(m0, last, sw0)
        wait_write(m1, last, sw1)

    return cls_gather


def kernel(node_features, edge_index, edge_features, W_msg, b_msg,
           W_ih, W_hh, b_ih, b_hh, W_c1, b_c1, W_c2, b_c2):
    n, d = node_features.shape
    e = edge_index.shape[1]
    de = edge_features.shape[1]
    c = W_c2.shape[1]
    assert e % _NW == 0 and (e // _NW) % 80 == 0 and n % _NS == 0

    chunk1 = 40
    nchunks1 = e // _NW // chunk1
    npairs1 = nchunks1 // 2
    s4 = edge_index[0].reshape(_NW, npairs1, 2, chunk1)
    t4 = edge_index[1].reshape(_NW, npairs1, 2, chunk1)
    idx_packed = jnp.stack(
        [s4[:, :, 0], t4[:, :, 0], s4[:, :, 1], t4[:, :, 1]], axis=2
    ).reshape(_NW * npairs1, 4, chunk1)

    chunk2 = 80
    nchunks2 = e // _NW // chunk2
    src = edge_index[0].reshape(_NW, nchunks2, chunk2)
    tgt = edge_index[1].reshape(_NW, nchunks2, chunk2)

    nblk = 2000
    a_tab, b_tab = pl.pallas_call(
        _node_tables_body,
        grid=(n // nblk,),
        in_specs=[
            pl.BlockSpec((nblk, d), lambda i: (i, 0)),
            pl.BlockSpec((d, d), lambda i: (0, 0)),
            pl.BlockSpec((d, d), lambda i: (0, 0)),
        ],
        out_specs=[
            pl.BlockSpec((nblk, d), lambda i: (i, 0)),
            pl.BlockSpec((nblk, d), lambda i: (i, 0)),
        ],
        out_shape=[
            jax.ShapeDtypeStruct((n, d), jnp.float32),
            jax.ShapeDtypeStruct((n, d), jnp.float32),
        ],
    )(node_features, W_msg[:d], W_msg[d:2 * d])

    eblk = 8000
    ce_tab, r_tab = pl.pallas_call(
        _edge_tables_body,
        grid=(e // eblk,),
        in_specs=[
            pl.BlockSpec((eblk, de), lambda i: (i, 0)),
            pl.BlockSpec((de, d), lambda i: (0, 0)),
            pl.BlockSpec((1, d), lambda i: (0, 0)),
            pl.BlockSpec((de, de), lambda i: (0, 0)),
            pl.BlockSpec((1, de), lambda i: (0, 0)),
        ],
        out_specs=[
            pl.BlockSpec((eblk, d), lambda i: (i, 0)),
            pl.BlockSpec((eblk, de), lambda i: (i, 0)),
        ],
        out_shape=[
            jax.ShapeDtypeStruct((e, d), jnp.float32),
            jax.ShapeDtypeStruct((e, de), jnp.float32),
        ],
    )(edge_features, W_msg[2 * d:], b_msg.reshape(1, d),
      W_c1[2 * d:], b_c1.reshape(1, de))

    msg_agg = _make_msg_agg(n, d, e, chunk=chunk1)
    agg2 = msg_agg(a_tab, b_tab, ce_tab, idx_packed)
    agg2 = agg2.reshape(2, msg_agg.n_pad, d)[:, :n, :]

    p_tab, q_tab = pl.pallas_call(
        _gru_body,
        grid=(n // nblk,),
        in_specs=[
            pl.BlockSpec((nblk, d), lambda i: (i, 0)),
            pl.BlockSpec((nblk, d), lambda i: (i, 0)),
            pl.BlockSpec((nblk, d), lambda i: (i, 0)),
            pl.BlockSpec((3 * d, d), lambda i: (0, 0)),
            pl.BlockSpec((3 * d, d), lambda i: (0, 0)),
            pl.BlockSpec((1, 3 * d), lambda i: (0, 0)),
            pl.BlockSpec((1, 3 * d), lambda i: (0, 0)),
            pl.BlockSpec((d, de), lambda i: (0, 0)),
            pl.BlockSpec((d, de), lambda i: (0, 0)),
        ],
        out_specs=[
            pl.BlockSpec((nblk, de), lambda i: (i, 0)),
            pl.BlockSpec((nblk, de), lambda i: (i, 0)),
        ],
        out_shape=[
            jax.ShapeDtypeStruct((n, de), jnp.float32),
            jax.ShapeDtypeStruct((n, de), jnp.float32),
        ],
    )(agg2[0], agg2[1], node_features, W_ih, W_hh,
      b_ih.reshape(1, 3 * d), b_hh.reshape(1, 3 * d),
      W_c1[:d], W_c1[d:2 * d])

    h = _make_cls_gather(n, de, e, chunk=chunk2)(p_tab, q_tab, r_tab, src, tgt)

    out = pl.pallas_call(
        _cls_out_body,
        grid=(e // eblk,),
        in_specs=[
            pl.BlockSpec((eblk, de), lambda i: (i, 0)),
            pl.BlockSpec((de, c), lambda i: (0, 0)),
            pl.BlockSpec((1, c), lambda i: (0, 0)),
        ],
        out_specs=pl.BlockSpec((eblk, c), lambda i: (i, 0)),
        out_shape=jax.ShapeDtypeStruct((e, c), jnp.float32),
    )(h, W_c2, b_c2.reshape(1, c))

    return out


# speedup vs baseline: 4.0576x; 1.1420x over previous
import functools

import jax
import jax.numpy as jnp
from jax import lax
from jax.experimental import pallas as pl
from jax.experimental.pallas import tpu as pltpu
from jax.experimental.pallas import tpu_sc as plsc

_NC = 2
_NS = 16
_NW = _NC * _NS
_LANES = 16


def _gelu_exp(x):
    e = jnp.exp(1.5957691216057308 * (x + 0.044715 * x * x * x))
    return x * e / (e + 1.0)


def _node_tables_body(nf_ref, w0_ref, w1_ref, a_ref, b_ref):
    x = nf_ref[...]
    a_ref[...] = jnp.dot(x, w0_ref[...], preferred_element_type=jnp.float32)
    b_ref[...] = jnp.dot(x, w1_ref[...], preferred_element_type=jnp.float32)


def _edge_tables_body(ef_ref, wce_ref, bce_ref, wr_ref, br_ref, ce_ref, r_ref):
    x = ef_ref[...]
    ce_ref[...] = jnp.dot(x, wce_ref[...], preferred_element_type=jnp.float32) + bce_ref[...]
    r_ref[...] = jnp.dot(x, wr_ref[...], preferred_element_type=jnp.float32) + br_ref[...]


def _gru_body(a0_ref, a1_ref, nf_ref, wih_ref, whh_ref, bih_ref, bhh_ref,
              wp_ref, wq_ref, p_ref, q_ref):
    d = nf_ref.shape[1]
    agg = a0_ref[...] + a1_ref[...]
    nf = nf_ref[...]
    gi = lax.dot_general(agg, wih_ref[...], (((1,), (1,)), ((), ())),
                         preferred_element_type=jnp.float32) + bih_ref[...]
    gh = lax.dot_general(nf, whh_ref[...], (((1,), (1,)), ((), ())),
                         preferred_element_type=jnp.float32) + bhh_ref[...]
    r = jax.nn.sigmoid(gi[:, :d] + gh[:, :d])
    z = jax.nn.sigmoid(gi[:, d:2 * d] + gh[:, d:2 * d])
    n = jnp.tanh(gi[:, 2 * d:] + r * gh[:, 2 * d:])
    nf_up = (1.0 - z) * n + z * nf
    p_ref[...] = jnp.dot(nf_up, wp_ref[...], preferred_element_type=jnp.float32)
    q_ref[...] = jnp.dot(nf_up, wq_ref[...], preferred_element_type=jnp.float32)


def _cls_out_body(h_ref, w_ref, b_ref, o_ref):
    o_ref[...] = jnp.dot(h_ref[...], w_ref[...],
                         preferred_element_type=jnp.float32) + b_ref[...]


def _make_msg_agg(n, d, e, chunk):
    ew = e // _NW
    nchunks = ew // chunk
    npairs = nchunks // 2
    zrows = chunk
    n_pad = -(-n // (_NS * zrows)) * (_NS * zrows)
    rows_per_tile = n_pad // _NS
    mesh = plsc.VectorSubcoreMesh(core_axis_name="c", subcore_axis_name="s")

    assert nchunks % 4 == 2 and nchunks >= 6 and chunk % 2 == 0
    buf_t = pltpu.VMEM((chunk, d), jnp.float32)
    idx_t = pltpu.VMEM((2, chunk), jnp.int32)

    @functools.partial(
        pl.kernel,
        mesh=mesh,
        out_type=jax.ShapeDtypeStruct((2 * n_pad, d), jnp.float32),
        scratch_types=[
            idx_t, idx_t,
            idx_t, idx_t,
            buf_t, buf_t, buf_t,
            buf_t, buf_t, buf_t,
            pltpu.VMEM_SHARED((n_pad, d), jnp.float32),
            pltpu.SemaphoreType.DMA,
            pltpu.SemaphoreType.DMA,
            pltpu.SemaphoreType.DMA,
            pltpu.SemaphoreType.DMA,
            pltpu.SemaphoreType.DMA,
            pltpu.SemaphoreType.DMA,
            pltpu.SemaphoreType.DMA,
            pltpu.SemaphoreType.DMA,
        ],
    )
    def msg_agg(a_hbm, b_hbm, ce_hbm, ei_hbm, out_hbm,
                isa, ita, isb, itb, ga0, gb0, gc0, ga1, gb1, gc1, agg_sh,
                sa0, sb0, sc0, sa1, sb1, sc1, sia, sib):
        cid = lax.axis_index("c")
        sid = lax.axis_index("s")
        wid = cid * _NS + sid
        base = wid * ew
        gsets = ((ga0, gb0, gc0, sa0, sb0, sc0), (ga1, gb1, gc1, sa1, sb1, sc1))

        def idx_start(sbuf, tbuf, sem, p):
            pltpu.async_copy(ei_hbm.at[0, wid, p], sbuf, sem)
            pltpu.async_copy(ei_hbm.at[1, wid, p], tbuf, sem)

        def idx_wait(sbuf, tbuf, sem, p):
            pltpu.make_async_copy(ei_hbm.at[0, wid, p], sbuf, sem).wait()
            pltpu.make_async_copy(ei_hbm.at[1, wid, p], tbuf, sem).wait()

        def fire(st, sbuf, tbuf, half, j):
            ga, gb, gc, sa, sb, sc = gsets[st]
            pltpu.async_copy(a_hbm.at[sbuf.at[half]], ga, sa)
            pltpu.async_copy(b_hbm.at[tbuf.at[half]], gb, sb)
            pltpu.async_copy(ce_hbm.at[pl.ds(base + j * chunk, chunk)], gc, sc)

        def wait(st, sbuf, tbuf, half, j):
            ga, gb, gc, sa, sb, sc = gsets[st]
            pltpu.make_async_copy(a_hbm.at[sbuf.at[half]], ga, sa).wait()
            pltpu.make_async_copy(b_hbm.at[tbuf.at[half]], gb, sb).wait()
            pltpu.make_async_copy(ce_hbm.at[pl.ds(base + j * chunk, chunk)], gc, sc).wait()

        def compute_scatter(st, tbuf, half):
            ga, gb, gc = gsets[st][:3]

            def row2(r2, carry):
                for u in range(2):
                    r = r2 * 2 + u
                    for c in range(d // _LANES):
                        s = pl.ds(_LANES * c, _LANES)
                        ga[r, s] = _gelu_exp(ga[r, s] + gb[r, s] + gc[r, s])
                return carry

            lax.fori_loop(0, chunk // 2, row2, 0)
            pltpu.sync_copy(ga, agg_sh.at[tbuf.at[half]], add=True)

        def zrow(i, carry):
            for c in range(d // _LANES):
                ga0[i, pl.ds(_LANES * c, _LANES)] = jnp.zeros((_LANES,), jnp.float32)
            return carry

        lax.fori_loop(0, chunk, zrow, 0)
        for t in range(rows_per_tile // zrows):
            pltpu.sync_copy(ga0, agg_sh.at[pl.ds(sid * rows_per_tile + t * zrows, zrows)])
        plsc.subcore_barrier()

        idx_start(isa, ita, sia, 0)
        idx_wait(isa, ita, sia, 0)
        fire(0, isa, ita, 0, 0)
        idx_start(isb, itb, sib, 1)

        def body(k, carry):
            a = 4 * k
            fire(1, isa, ita, 1, a + 1)
            wait(0, isa, ita, 0, a)
            compute_scatter(0, ita, 0)
            idx_wait(isb, itb, sib, 2 * k + 1)
            fire(0, isb, itb, 0, a + 2)
            wait(1, isa, ita, 1, a + 1)
            compute_scatter(1, ita, 1)
            idx_start(isa, ita, sia, 2 * k + 2)
            fire(1, isb, itb, 1, a + 3)
            wait(0, isb, itb, 0, a + 2)
            compute_scatter(0, itb, 0)
            idx_wait(isa, ita, sia, 2 * k + 2)
            fire(0, isa, ita, 0, a + 4)
            wait(1, isb, itb, 1, a + 3)
            compute_scatter(1, itb, 1)
            idx_start(isb, itb, sib, jnp.minimum(2 * k + 3, npairs - 1))
            return carry

        lax.fori_loop(0, (npairs - 1) // 2, body, 0)
        last = nchunks - 2
        fire(1, isa, ita, 1, last + 1)
        wait(0, isa, ita, 0, last)
        compute_scatter(0, ita, 0)
        wait(1, isa, ita, 1, last + 1)
        compute_scatter(1, ita, 1)
        idx_wait(isb, itb, sib, npairs - 1)

        plsc.subcore_barrier()
        for t in range(rows_per_tile // zrows):
            sl = pl.ds(sid * rows_per_tile + t * zrows, zrows)
            dst = pl.ds(cid * n_pad + sid * rows_per_tile + t * zrows, zrows)
            pltpu.sync_copy(agg_sh.at[sl], out_hbm.at[dst])

    msg_agg.n_pad = n_pad
    return msg_agg


def _make_cls_gather(n, de, e, chunk):
    ew = e // _NW
    nchunks = ew // chunk
    assert nchunks % 2 == 1 and nchunks >= 3
    mesh = plsc.VectorSubcoreMesh(core_axis_name="c", subcore_axis_name="s")
    buf_t = pltpu.VMEM((chunk, de), jnp.float32)

    @functools.partial(
        pl.kernel,
        mesh=mesh,
        out_type=jax.ShapeDtypeStruct((e, de), jnp.float32),
        scratch_types=[
            pltpu.VMEM((nchunks, chunk), jnp.int32),
            pltpu.VMEM((nchunks, chunk), jnp.int32),
            buf_t, buf_t, buf_t,
            buf_t, buf_t, buf_t,
            buf_t, buf_t,
            pltpu.SemaphoreType.DMA,
            pltpu.SemaphoreType.DMA,
            pltpu.SemaphoreType.DMA,
            pltpu.SemaphoreType.DMA,
            pltpu.SemaphoreType.DMA,
            pltpu.SemaphoreType.DMA,
            pltpu.SemaphoreType.DMA,
            pltpu.SemaphoreType.DMA,
        ],
        compiler_params=pltpu.CompilerParams(use_tc_tiling_on_sc=False),
    )
    def cls_gather(p_hbm, q_hbm, r_hbm, ei_hbm, h_hbm,
                   sslab, tslab, gp0, gq0, gr0, gp1, gq1, gr1, m0, m1,
                   sp0, sq0, sr0, sp1, sq1, sr1, sw0, sw1):
        cid = lax.axis_index("c")
        sid = lax.axis_index("s")
        wid = cid * _NS + sid
        base = wid * ew
        gsets = ((gp0, gq0, gr0, sp0, sq0, sr0), (gp1, gq1, gr1, sp1, sq1, sr1))

        def fire_gathers(st, j):
            gp, gq, gr, sp, sq, sr = gsets[st]
            pltpu.async_copy(p_hbm.at[sslab.at[j]], gp, sp)
            pltpu.async_copy(q_hbm.at[tslab.at[j]], gq, sq)
            pltpu.async_copy(r_hbm.at[pl.ds(base + j * chunk, chunk)], gr, sr)

        def wait_gathers(st, j):
            gp, gq, gr, sp, sq, sr = gsets[st]
            pltpu.make_async_copy(p_hbm.at[sslab.at[j]], gp, sp).wait()
            pltpu.make_async_copy(q_hbm.at[tslab.at[j]], gq, sq).wait()
            pltpu.make_async_copy(r_hbm.at[pl.ds(base + j * chunk, chunk)], gr, sr).wait()

        def compute(st, m):
            gp, gq, gr = gsets[st][:3]

            def rows8(r8, carry):
                for u in range(8):
                    r = r8 * 8 + u
                    s = pl.ds(0, _LANES)
                    m[r, s] = _gelu_exp(gp[r, s] + gq[r, s] + gr[r, s])
                return carry

            lax.fori_loop(0, chunk // 8, rows8, 0)

        pltpu.sync_copy(ei_hbm.at[0, wid], sslab)
        pltpu.sync_copy(ei_hbm.at[1, wid], tslab)

        pltpu.async_copy(m0, h_hbm.at[pl.ds(base, chunk)], sw0)
        pltpu.async_copy(m1, h_hbm.at[pl.ds(base + chunk, chunk)], sw1)
        fire_gathers(0, 0)

        def wait_write(m, j, sem):
            pltpu.make_async_copy(m, h_hbm.at[pl.ds(base + j * chunk, chunk)], sem).wait()

        def body(j2, carry):
            a = 2 * j2
            b = a + 1
            fire_gathers(1, b)
            wait_gathers(0, a)
            wait_write(m0, a, sw0)
            compute(0, m0)
            pltpu.async_copy(m0, h_hbm.at[pl.ds(base + a * chunk, chunk)], sw0)
            fire_gathers(0, a + 2)
            wait_gathers(1, b)
            wait_write(m1, b, sw1)
            compute(1, m1)
            pltpu.async_copy(m1, h_hbm.at[pl.ds(base + b * chunk, chunk)], sw1)
            return carry

        lax.fori_loop(0, nchunks // 2, body, 0)
        last = nchunks - 1
        wait_gathers(0, last)
        wait_write(m0, last, sw0)
        compute(0, m0)
        pltpu.async_copy(m0, h_hbm.at[pl.ds(base + last * chunk, chunk)], sw0)
        wait_write(m0, last, sw0)
        wait_write(m1, last, sw1)

    return cls_gather


def kernel(node_features, edge_index, edge_features, W_msg, b_msg,
           W_ih, W_hh, b_ih, b_hh, W_c1, b_c1, W_c2, b_c2):
    n, d = node_features.shape
    e = edge_index.shape[1]
    de = edge_features.shape[1]
    c = W_c2.shape[1]
    assert e % _NW == 0 and (e // _NW) % 80 == 0 and n % _NS == 0

    chunk1 = 40
    npairs1 = e // _NW // chunk1 // 2
    ei5 = edge_index.reshape(2, _NW, npairs1, 2, chunk1)
    chunk2 = 80
    nchunks2 = e // _NW // chunk2
    ei4 = edge_index.reshape(2, _NW, nchunks2, chunk2)

    nblk = 2000
    a_tab, b_tab = pl.pallas_call(
        _node_tables_body,
        grid=(n // nblk,),
        in_specs=[
            pl.BlockSpec((nblk, d), lambda i: (i, 0)),
            pl.BlockSpec((d, d), lambda i: (0, 0)),
            pl.BlockSpec((d, d), lambda i: (0, 0)),
        ],
        out_specs=[
            pl.BlockSpec((nblk, d), lambda i: (i, 0)),
            pl.BlockSpec((nblk, d), lambda i: (i, 0)),
        ],
        out_shape=[
            jax.ShapeDtypeStruct((n, d), jnp.float32),
            jax.ShapeDtypeStruct((n, d), jnp.float32),
        ],
    )(node_features, W_msg[:d], W_msg[d:2 * d])

    eblk = 16000
    ce_tab, r_tab = pl.pallas_call(
        _edge_tables_body,
        grid=(e // eblk,),
        in_specs=[
            pl.BlockSpec((eblk, de), lambda i: (i, 0)),
            pl.BlockSpec((de, d), lambda i: (0, 0)),
            pl.BlockSpec((1, d), lambda i: (0, 0)),
            pl.BlockSpec((de, de), lambda i: (0, 0)),
            pl.BlockSpec((1, de), lambda i: (0, 0)),
        ],
        out_specs=[
            pl.BlockSpec((eblk, d), lambda i: (i, 0)),
            pl.BlockSpec((eblk, de), lambda i: (i, 0)),
        ],
        out_shape=[
            jax.ShapeDtypeStruct((e, d), jnp.float32),
            jax.ShapeDtypeStruct((e, de), jnp.float32),
        ],
    )(edge_features, W_msg[2 * d:], b_msg.reshape(1, d),
      W_c1[2 * d:], b_c1.reshape(1, de))

    msg_agg = _make_msg_agg(n, d, e, chunk=chunk1)
    agg2 = msg_agg(a_tab, b_tab, ce_tab, ei5)
    agg2 = agg2.reshape(2, msg_agg.n_pad, d)[:, :n, :]

    p_tab, q_tab = pl.pallas_call(
        _gru_body,
        grid=(n // nblk,),
        in_specs=[
            pl.BlockSpec((nblk, d), lambda i: (i, 0)),
            pl.BlockSpec((nblk, d), lambda i: (i, 0)),
            pl.BlockSpec((nblk, d), lambda i: (i, 0)),
            pl.BlockSpec((3 * d, d), lambda i: (0, 0)),
            pl.BlockSpec((3 * d, d), lambda i: (0, 0)),
            pl.BlockSpec((1, 3 * d), lambda i: (0, 0)),
            pl.BlockSpec((1, 3 * d), lambda i: (0, 0)),
            pl.BlockSpec((d, de), lambda i: (0, 0)),
            pl.BlockSpec((d, de), lambda i: (0, 0)),
        ],
        out_specs=[
            pl.BlockSpec((nblk, de), lambda i: (i, 0)),
            pl.BlockSpec((nblk, de), lambda i: (i, 0)),
        ],
        out_shape=[
            jax.ShapeDtypeStruct((n, de), jnp.float32),
            jax.ShapeDtypeStruct((n, de), jnp.float32),
        ],
    )(agg2[0], agg2[1], node_features, W_ih, W_hh,
      b_ih.reshape(1, 3 * d), b_hh.reshape(1, 3 * d),
      W_c1[:d], W_c1[d:2 * d])

    h = _make_cls_gather(n, de, e, chunk=chunk2)(p_tab, q_tab, r_tab, ei4)

    out = pl.pallas_call(
        _cls_out_body,
        grid=(e // eblk,),
        in_specs=[
            pl.BlockSpec((eblk, de), lambda i: (i, 0)),
            pl.BlockSpec((de, c), lambda i: (0, 0)),
            pl.BlockSpec((1, c), lambda i: (0, 0)),
        ],
        out_specs=pl.BlockSpec((eblk, c), lambda i: (i, 0)),
        out_shape=jax.ShapeDtypeStruct((e, c), jnp.float32),
    )(h, W_c2, b_c2.reshape(1, c))

    return out


# speedup vs baseline: 4.3985x; 1.0840x over previous
import functools

import jax
import jax.numpy as jnp
from jax import lax
from jax.experimental import pallas as pl
from jax.experimental.pallas import tpu as pltpu
from jax.experimental.pallas import tpu_sc as plsc

_NC = 2
_NS = 16
_NW = _NC * _NS
_LANES = 16


def _gelu_exp(x):
    e = jnp.exp(1.5957691216057308 * (x + 0.044715 * x * x * x))
    return x * e / (e + 1.0)


def _node_tables_body(nf_ref, w0_ref, w1_ref, a_ref, b_ref):
    x = nf_ref[...]
    a_ref[...] = jnp.dot(x, w0_ref[...], preferred_element_type=jnp.float32)
    b_ref[...] = jnp.dot(x, w1_ref[...], preferred_element_type=jnp.float32)


def _edge_tables_body(ef_ref, wce_ref, bce_ref, wr_ref, br_ref, ce_ref, r_ref):
    x = ef_ref[...]
    ce_ref[...] = jnp.dot(x, wce_ref[...], preferred_element_type=jnp.float32) + bce_ref[...]
    r_ref[...] = jnp.dot(x, wr_ref[...], preferred_element_type=jnp.float32) + br_ref[...]


def _gru_body(a0_ref, a1_ref, nf_ref, wih_ref, whh_ref, bih_ref, bhh_ref,
              wp_ref, wq_ref, p_ref, q_ref):
    d = nf_ref.shape[1]
    agg = a0_ref[...] + a1_ref[...]
    nf = nf_ref[...]
    gi = lax.dot_general(agg, wih_ref[...], (((1,), (1,)), ((), ())),
                         preferred_element_type=jnp.float32) + bih_ref[...]
    gh = lax.dot_general(nf, whh_ref[...], (((1,), (1,)), ((), ())),
                         preferred_element_type=jnp.float32) + bhh_ref[...]
    r = jax.nn.sigmoid(gi[:, :d] + gh[:, :d])
    z = jax.nn.sigmoid(gi[:, d:2 * d] + gh[:, d:2 * d])
    n = jnp.tanh(gi[:, 2 * d:] + r * gh[:, 2 * d:])
    nf_up = (1.0 - z) * n + z * nf
    p_ref[...] = jnp.dot(nf_up, wp_ref[...], preferred_element_type=jnp.float32)
    q_ref[...] = jnp.dot(nf_up, wq_ref[...], preferred_element_type=jnp.float32)


def _cls_out_body(h_ref, w_ref, b_ref, o_ref):
    o_ref[...] = jnp.dot(h_ref[...], w_ref[...],
                         preferred_element_type=jnp.float32) + b_ref[...]


def _make_msg_agg(n, d, e, chunk):
    ew = e // _NW
    nchunks = ew // chunk
    npairs = nchunks // 2
    zrows = chunk
    n_pad = -(-n // (_NS * zrows)) * (_NS * zrows)
    rows_per_tile = n_pad // _NS
    mesh = plsc.VectorSubcoreMesh(core_axis_name="c", subcore_axis_name="s")

    assert nchunks % 4 == 2 and nchunks >= 6 and chunk % 2 == 0
    buf_t = pltpu.VMEM((chunk, d), jnp.float32)
    idx_t = pltpu.VMEM((2, chunk), jnp.int32)

    @functools.partial(
        pl.kernel,
        mesh=mesh,
        out_type=jax.ShapeDtypeStruct((2 * n_pad, d), jnp.float32),
        scratch_types=[
            idx_t, idx_t,
            idx_t, idx_t,
            buf_t, buf_t, buf_t,
            buf_t, buf_t, buf_t,
            pltpu.VMEM_SHARED((n_pad, d), jnp.float32),
            pltpu.SemaphoreType.DMA,
            pltpu.SemaphoreType.DMA,
            pltpu.SemaphoreType.DMA,
            pltpu.SemaphoreType.DMA,
            pltpu.SemaphoreType.DMA,
            pltpu.SemaphoreType.DMA,
            pltpu.SemaphoreType.DMA,
            pltpu.SemaphoreType.DMA,
        ],
    )
    def msg_agg(a_hbm, b_hbm, ce_hbm, ei_hbm, out_hbm,
                isa, ita, isb, itb, ga0, gb0, gc0, ga1, gb1, gc1, agg_sh,
                sa0, sb0, sc0, sa1, sb1, sc1, sia, sib):
        cid = lax.axis_index("c")
        sid = lax.axis_index("s")
        wid = cid * _NS + sid
        base = wid * ew
        gsets = ((ga0, gb0, gc0, sa0, sb0, sc0), (ga1, gb1, gc1, sa1, sb1, sc1))

        def idx_start(sbuf, tbuf, sem, p):
            pltpu.async_copy(ei_hbm.at[0, wid, p], sbuf, sem)
            pltpu.async_copy(ei_hbm.at[1, wid, p], tbuf, sem)

        def idx_wait(sbuf, tbuf, sem, p):
            pltpu.make_async_copy(ei_hbm.at[0, wid, p], sbuf, sem).wait()
            pltpu.make_async_copy(ei_hbm.at[1, wid, p], tbuf, sem).wait()

        def fire(st, sbuf, tbuf, half, j):
            ga, gb, gc, sa, sb, sc = gsets[st]
            pltpu.async_copy(a_hbm.at[sbuf.at[half]], ga, sa)
            pltpu.async_copy(b_hbm.at[tbuf.at[half]], gb, sb)
            pltpu.async_copy(ce_hbm.at[pl.ds(base + j * chunk, chunk)], gc, sc)

        def wait(st, sbuf, tbuf, half, j):
            ga, gb, gc, sa, sb, sc = gsets[st]
            pltpu.make_async_copy(a_hbm.at[sbuf.at[half]], ga, sa).wait()
            pltpu.make_async_copy(b_hbm.at[tbuf.at[half]], gb, sb).wait()
            pltpu.make_async_copy(ce_hbm.at[pl.ds(base + j * chunk, chunk)], gc, sc).wait()

        def compute_scatter(st, tbuf, half):
            ga, gb, gc = gsets[st][:3]

            def row2(r2, carry):
                for u in range(2):
                    r = r2 * 2 + u
                    for c in range(d // _LANES):
                        s = pl.ds(_LANES * c, _LANES)
                        ga[r, s] = _gelu_exp(ga[r, s] + gb[r, s] + gc[r, s])
                return carry

            lax.fori_loop(0, chunk // 2, row2, 0)
            pltpu.sync_copy(ga, agg_sh.at[tbuf.at[half]], add=True)

        def zrow(i, carry):
            for c in range(d // _LANES):
                ga0[i, pl.ds(_LANES * c, _LANES)] = jnp.zeros((_LANES,), jnp.float32)
            return carry

        lax.fori_loop(0, chunk, zrow, 0)
        for t in range(rows_per_tile // zrows):
            pltpu.sync_copy(ga0, agg_sh.at[pl.ds(sid * rows_per_tile + t * zrows, zrows)])
        plsc.subcore_barrier()

        idx_start(isa, ita, sia, 0)
        idx_wait(isa, ita, sia, 0)
        fire(0, isa, ita, 0, 0)
        idx_start(isb, itb, sib, 1)

        def body(k, carry):
            a = 4 * k
            fire(1, isa, ita, 1, a + 1)
            wait(0, isa, ita, 0, a)
            compute_scatter(0, ita, 0)
            idx_wait(isb, itb, sib, 2 * k + 1)
            fire(0, isb, itb, 0, a + 2)
            wait(1, isa, ita, 1, a + 1)
            compute_scatter(1, ita, 1)
            idx_start(isa, ita, sia, 2 * k + 2)
            fire(1, isb, itb, 1, a + 3)
            wait(0, isb, itb, 0, a + 2)
            compute_scatter(0, itb, 0)
            idx_wait(isa, ita, sia, 2 * k + 2)
            fire(0, isa, ita, 0, a + 4)
            wait(1, isb, itb, 1, a + 3)
            compute_scatter(1, itb, 1)
            idx_start(isb, itb, sib, jnp.minimum(2 * k + 3, npairs - 1))
            return carry

        lax.fori_loop(0, (npairs - 1) // 2, body, 0)
        last = nchunks - 2
        fire(1, isa, ita, 1, last + 1)
        wait(0, isa, ita, 0, last)
        compute_scatter(0, ita, 0)
        wait(1, isa, ita, 1, last + 1)
        compute_scatter(1, ita, 1)
        idx_wait(isb, itb, sib, npairs - 1)

        plsc.subcore_barrier()
        for t in range(rows_per_tile // zrows):
            sl = pl.ds(sid * rows_per_tile + t * zrows, zrows)
            dst = pl.ds(cid * n_pad + sid * rows_per_tile + t * zrows, zrows)
            pltpu.sync_copy(agg_sh.at[sl], out_hbm.at[dst])

    msg_agg.n_pad = n_pad
    return msg_agg


def _make_cls_gather(n, de, e, chunk):
    ew = e // _NW
    nchunks = ew // chunk
    assert nchunks % 2 == 1 and nchunks >= 3
    assert chunk % 8 == 0 and ew % 8 == 0 and (128 % de) == 0
    epr = 128 // de
    crows = chunk // epr
    mesh = plsc.VectorSubcoreMesh(core_axis_name="c", subcore_axis_name="s")
    buf_t = pltpu.VMEM((chunk, de), jnp.float32)
    hbuf_t = pltpu.VMEM((crows, 128), jnp.float32)

    @functools.partial(
        pl.kernel,
        mesh=mesh,
        out_type=jax.ShapeDtypeStruct((e // epr, 128), jnp.float32),
        scratch_types=[
            pltpu.VMEM((nchunks, chunk), jnp.int32),
            pltpu.VMEM((nchunks, chunk), jnp.int32),
            buf_t, buf_t, buf_t,
            buf_t, buf_t, buf_t,
            hbuf_t, hbuf_t,
            pltpu.SemaphoreType.DMA,
            pltpu.SemaphoreType.DMA,
            pltpu.SemaphoreType.DMA,
            pltpu.SemaphoreType.DMA,
            pltpu.SemaphoreType.DMA,
            pltpu.SemaphoreType.DMA,
            pltpu.SemaphoreType.DMA,
            pltpu.SemaphoreType.DMA,
        ],
        compiler_params=pltpu.CompilerParams(use_tc_tiling_on_sc=False),
    )
    def cls_gather(p_hbm, q_hbm, r_hbm, ei_hbm, h_hbm,
                   sslab, tslab, gp0, gq0, gr0, gp1, gq1, gr1, m0, m1,
                   sp0, sq0, sr0, sp1, sq1, sr1, sw0, sw1):
        cid = lax.axis_index("c")
        sid = lax.axis_index("s")
        wid = cid * _NS + sid
        base = wid * ew
        rbase = wid * (ew // epr)
        gsets = ((gp0, gq0, gr0, sp0, sq0, sr0), (gp1, gq1, gr1, sp1, sq1, sr1))

        def fire_gathers(st, j):
            gp, gq, gr, sp, sq, sr = gsets[st]
            pltpu.async_copy(p_hbm.at[sslab.at[j]], gp, sp)
            pltpu.async_copy(q_hbm.at[tslab.at[j]], gq, sq)
            pltpu.async_copy(r_hbm.at[pl.ds(base + j * chunk, chunk)], gr, sr)

        def wait_gathers(st, j):
            gp, gq, gr, sp, sq, sr = gsets[st]
            pltpu.make_async_copy(p_hbm.at[sslab.at[j]], gp, sp).wait()
            pltpu.make_async_copy(q_hbm.at[tslab.at[j]], gq, sq).wait()
            pltpu.make_async_copy(r_hbm.at[pl.ds(base + j * chunk, chunk)], gr, sr).wait()

        def compute(st, m):
            gp, gq, gr = gsets[st][:3]

            def rows(pr, carry):
                for u in range(epr):
                    r = pr * epr + u
                    s = pl.ds(0, de)
                    m[pr, pl.ds(de * u, de)] = _gelu_exp(gp[r, s] + gq[r, s] + gr[r, s])
                return carry

            lax.fori_loop(0, crows, rows, 0)

        pltpu.sync_copy(ei_hbm.at[0, wid], sslab)
        pltpu.sync_copy(ei_hbm.at[1, wid], tslab)

        pltpu.async_copy(m0, h_hbm.at[pl.ds(rbase, crows)], sw0)
        pltpu.async_copy(m1, h_hbm.at[pl.ds(rbase + crows, crows)], sw1)
        fire_gathers(0, 0)

        def wait_write(m, j, sem):
            pltpu.make_async_copy(m, h_hbm.at[pl.ds(rbase + j * crows, crows)], sem).wait()

        def body(j2, carry):
            a = 2 * j2
            b = a + 1
            fire_gathers(1, b)
            wait_gathers(0, a)
            wait_write(m0, a, sw0)
            compute(0, m0)
            pltpu.async_copy(m0, h_hbm.at[pl.ds(rbase + a * crows, crows)], sw0)
            fire_gathers(0, a + 2)
            wait_gathers(1, b)
            wait_write(m1, b, sw1)
            compute(1, m1)
            pltpu.async_copy(m1, h_hbm.at[pl.ds(rbase + b * crows, crows)], sw1)
            return carry

        lax.fori_loop(0, nchunks // 2, body, 0)
        last = nchunks - 1
        wait_gathers(0, last)
        wait_write(m0, last, sw0)
        compute(0, m0)
        pltpu.async_copy(m0, h_hbm.at[pl.ds(rbase + last * crows, crows)], sw0)
        wait_write(m0, last, sw0)
        wait_write(m1, last, sw1)

    return cls_gather


def kernel(node_features, edge_index, edge_features, W_msg, b_msg,
           W_ih, W_hh, b_ih, b_hh, W_c1, b_c1, W_c2, b_c2):
    n, d = node_features.shape
    e = edge_index.shape[1]
    de = edge_features.shape[1]
    c = W_c2.shape[1]
    assert e % _NW == 0 and (e // _NW) % 80 == 0 and n % _NS == 0

    chunk1 = 40
    npairs1 = e // _NW // chunk1 // 2
    ei5 = edge_index.reshape(2, _NW, npairs1, 2, chunk1)
    chunk2 = 80
    nchunks2 = e // _NW // chunk2
    ei4 = edge_index.reshape(2, _NW, nchunks2, chunk2)

    nblk = 2000
    a_tab, b_tab = pl.pallas_call(
        _node_tables_body,
        grid=(n // nblk,),
        in_specs=[
            pl.BlockSpec((nblk, d), lambda i: (i, 0)),
            pl.BlockSpec((d, d), lambda i: (0, 0)),
            pl.BlockSpec((d, d), lambda i: (0, 0)),
        ],
        out_specs=[
            pl.BlockSpec((nblk, d), lambda i: (i, 0)),
            pl.BlockSpec((nblk, d), lambda i: (i, 0)),
        ],
        out_shape=[
            jax.ShapeDtypeStruct((n, d), jnp.float32),
            jax.ShapeDtypeStruct((n, d), jnp.float32),
        ],
    )(node_features, W_msg[:d], W_msg[d:2 * d])

    eblk = 16000
    ce_tab, r_tab = pl.pallas_call(
        _edge_tables_body,
        grid=(e // eblk,),
        in_specs=[
            pl.BlockSpec((eblk, de), lambda i: (i, 0)),
            pl.BlockSpec((de, d), lambda i: (0, 0)),
            pl.BlockSpec((1, d), lambda i: (0, 0)),
            pl.BlockSpec((de, de), lambda i: (0, 0)),
            pl.BlockSpec((1, de), lambda i: (0, 0)),
        ],
        out_specs=[
            pl.BlockSpec((eblk, d), lambda i: (i, 0)),
            pl.BlockSpec((eblk, de), lambda i: (i, 0)),
        ],
        out_shape=[
            jax.ShapeDtypeStruct((e, d), jnp.float32),
            jax.ShapeDtypeStruct((e, de), jnp.float32),
        ],
    )(edge_features, W_msg[2 * d:], b_msg.reshape(1, d),
      W_c1[2 * d:], b_c1.reshape(1, de))

    msg_agg = _make_msg_agg(n, d, e, chunk=chunk1)
    agg2 = msg_agg(a_tab, b_tab, ce_tab, ei5)
    agg2 = agg2.reshape(2, msg_agg.n_pad, d)[:, :n, :]

    p_tab, q_tab = pl.pallas_call(
        _gru_body,
        grid=(n // nblk,),
        in_specs=[
            pl.BlockSpec((nblk, d), lambda i: (i, 0)),
            pl.BlockSpec((nblk, d), lambda i: (i, 0)),
            pl.BlockSpec((nblk, d), lambda i: (i, 0)),
            pl.BlockSpec((3 * d, d), lambda i: (0, 0)),
            pl.BlockSpec((3 * d, d), lambda i: (0, 0)),
            pl.BlockSpec((1, 3 * d), lambda i: (0, 0)),
            pl.BlockSpec((1, 3 * d), lambda i: (0, 0)),
            pl.BlockSpec((d, de), lambda i: (0, 0)),
            pl.BlockSpec((d, de), lambda i: (0, 0)),
        ],
        out_specs=[
            pl.BlockSpec((nblk, de), lambda i: (i, 0)),
            pl.BlockSpec((nblk, de), lambda i: (i, 0)),
        ],
        out_shape=[
            jax.ShapeDtypeStruct((n, de), jnp.float32),
            jax.ShapeDtypeStruct((n, de), jnp.float32),
        ],
    )(agg2[0], agg2[1], node_features, W_ih, W_hh,
      b_ih.reshape(1, 3 * d), b_hh.reshape(1, 3 * d),
      W_c1[:d], W_c1[d:2 * d])

    epr = 128 // de
    h = _make_cls_gather(n, de, e, chunk=chunk2)(p_tab, q_tab, r_tab, ei4)

    w_bd = jnp.kron(jnp.eye(epr, dtype=jnp.float32), W_c2)
    b_pk = jnp.tile(b_c2, epr).reshape(1, epr * c)
    hblk = 4000
    out_pk = pl.pallas_call(
        _cls_out_body,
        grid=(e // epr // hblk,),
        in_specs=[
            pl.BlockSpec((hblk, 128), lambda i: (i, 0)),
            pl.BlockSpec((128, epr * c), lambda i: (0, 0)),
            pl.BlockSpec((1, epr * c), lambda i: (0, 0)),
        ],
        out_specs=pl.BlockSpec((hblk, epr * c), lambda i: (i, 0)),
        out_shape=jax.ShapeDtypeStruct((e // epr, epr * c), jnp.float32),
    )(h, w_bd, b_pk)

    return out_pk.reshape(e, c)
